# baseline (speedup 1.0000x reference)
import functools

import jax
import jax.numpy as jnp
from jax import lax
from jax.experimental import pallas as pl
from jax.experimental.pallas import tpu as pltpu
from jax.experimental.pallas import tpu_sc as plsc

_F = 26
_D = 50
_B = 16384
_L = 16
_NC = 2
_NS = 16
_NW = _NC * _NS
_COLS = _B // _NW
_CSTEPS = _COLS // _L
_TAB = _F * _D
_TPAD = ((_TAB + _L - 1) // _L) * _L
_DS = 56
_TAB56 = _F * _DS + _L
_FPAD = 32
_SLAB = _F * _COLS


def _make_sc_kernel():
    mesh = plsc.VectorSubcoreMesh(core_axis_name="c", subcore_axis_name="s")

    @functools.partial(
        pl.kernel,
        mesh=mesh,
        out_type=jax.ShapeDtypeStruct((_F, _B), jnp.float32),
        compiler_params=pltpu.CompilerParams(needs_layout_passes=False,
                                             skip_device_barrier=True),
        scratch_types=[
            pltpu.VMEM((_SLAB,), jnp.int32),
            pltpu.VMEM((_SLAB,), jnp.float32),
            pltpu.VMEM((_TPAD,), jnp.float32),
            pltpu.VMEM((_TAB56,), jnp.float32),
            pltpu.VMEM((_FPAD,), jnp.float32),
            pltpu.VMEM((_FPAD,), jnp.float32),
            pltpu.SemaphoreType.DMA,
            pltpu.SemaphoreType.DMA,
            pltpu.SemaphoreType.DMA,
        ],
    )
    def sc_kernel(idx_hbm, bias_hbm, mean_hbm, norm_hbm, out_hbm,
                  idx_v, out_v, tab_v, tab56_v, mean_v, norm_v,
                  isem, osem, bsem):
        wid = lax.axis_index("s") * _NC + lax.axis_index("c")
        col0 = wid * _COLS

        idx_h = [
            pltpu.async_copy(idx_hbm.at[f, pl.ds(col0, _COLS)],
                             idx_v.at[pl.ds(f * _COLS, _COLS)], isem)
            for f in range(_F)
        ]
        bias_h = pltpu.async_copy(bias_hbm, tab_v.at[pl.ds(0, _TAB)], bsem)
        pltpu.sync_copy(mean_hbm, mean_v.at[pl.ds(0, _F)])
        pltpu.sync_copy(norm_hbm, norm_v.at[pl.ds(0, _F)])
        bias_h.wait()

        lane = lax.iota(jnp.int32, _L)

        @plsc.parallel_loop(0, _TPAD // _L, unroll=2)
        def prescale(t):
            flat = lane + t * _L
            fidx = flat // _D
            m = plsc.load_gather(mean_v, [fidx])
            nrm = plsc.load_gather(norm_v, [fidx])
            val = (tab_v[pl.ds(t * _L, _L)] - m) / nrm
            plsc.store_scatter(tab56_v, [flat + fidx * (_DS - _D)], val)

        for h in idx_h:
            h.wait()

        tab_f = [tab56_v.at[pl.ds(f * _DS, _DS)] for f in range(_F)]

        @plsc.parallel_loop(0, _CSTEPS, unroll=4)
        def gather_loop(c):
            for f in range(_F):
                sl = pl.ds(f * _COLS + c * _L, _L)
                out_v[sl] = plsc.load_gather(tab_f[f], [idx_v[sl]])

        out_h = [
            pltpu.async_copy(out_v.at[pl.ds(f * _COLS, _COLS)],
                             out_hbm.at[f, pl.ds(col0, _COLS)], osem)
            for f in range(_F)
        ]
        for h in out_h:
            h.wait()

    return sc_kernel


_make_sc_kernel = functools.cache(_make_sc_kernel)


def kernel(inputs, categ_bias, moving_mean, moving_norm):
    idx_t = jnp.transpose(inputs)
    bias_flat = jnp.reshape(categ_bias, (_TAB,))
    mean_flat = jnp.reshape(moving_mean, (_F,))
    norm_flat = jnp.reshape(moving_norm, (_F,))
    out_t = _make_sc_kernel()(idx_t, bias_flat, mean_flat, norm_flat)
    return jnp.transpose(out_t)


# speedup vs baseline: 1.0471x; 1.0471x over previous
import functools

import jax
import jax.numpy as jnp
from jax import lax
from jax.experimental import pallas as pl
from jax.experimental.pallas import tpu as pltpu
from jax.experimental.pallas import tpu_sc as plsc

_F = 26
_D = 50
_B = 16384
_L = 16
_NC = 2
_NS = 16
_NW = _NC * _NS
_COLS = _B // _NW
_CSTEPS = _COLS // _L
_TAB = _F * _D
_TPAD = ((_TAB + _L - 1) // _L) * _L
_DS = 56
_TAB56 = _F * _DS + _L
_FPAD = 32
_SLAB = _F * _COLS


def _make_sc_kernel():
    mesh = plsc.VectorSubcoreMesh(core_axis_name="c", subcore_axis_name="s")

    @functools.partial(
        pl.kernel,
        mesh=mesh,
        out_type=jax.ShapeDtypeStruct((_F, _B), jnp.float32),
        compiler_params=pltpu.CompilerParams(needs_layout_passes=False,
                                             skip_device_barrier=True),
        scratch_types=[
            pltpu.VMEM((_SLAB,), jnp.int32),
            pltpu.VMEM((_SLAB,), jnp.float32),
            pltpu.VMEM((_TPAD,), jnp.float32),
            pltpu.VMEM((_TAB56,), jnp.float32),
            pltpu.VMEM((_FPAD,), jnp.float32),
            pltpu.VMEM((_FPAD,), jnp.float32),
            pltpu.SemaphoreType.DMA,
            pltpu.SemaphoreType.DMA,
            pltpu.SemaphoreType.DMA,
        ],
    )
    def sc_kernel(idx_hbm, bias_hbm, mean_hbm, norm_hbm, out_hbm,
                  idx_v, out_v, tab_v, tab56_v, mean_v, norm_v,
                  isem, osem, bsem):
        wid = lax.axis_index("s") * _NC + lax.axis_index("c")
        col0 = wid * _COLS

        idx_h = [
            pltpu.async_copy(idx_hbm.at[f, pl.ds(col0, _COLS)],
                             idx_v.at[pl.ds(f * _COLS, _COLS)], isem)
            for f in range(_F)
        ]
        bias_h = pltpu.async_copy(bias_hbm, tab_v.at[pl.ds(0, _TAB)], bsem)
        pltpu.sync_copy(mean_hbm, mean_v.at[pl.ds(0, _F)])
        pltpu.sync_copy(norm_hbm, norm_v.at[pl.ds(0, _F)])
        bias_h.wait()

        lane = lax.iota(jnp.int32, _L)

        @plsc.parallel_loop(0, _TPAD // _L, unroll=2)
        def prescale(t):
            flat = lane + t * _L
            fidx = flat // _D
            m = plsc.load_gather(mean_v, [fidx])
            nrm = plsc.load_gather(norm_v, [fidx])
            val = (tab_v[pl.ds(t * _L, _L)] - m) / nrm
            plsc.store_scatter(tab56_v, [flat + fidx * (_DS - _D)], val)

        for h in idx_h:
            h.wait()

        tab_f = [tab56_v.at[pl.ds(f * _DS, _DS)] for f in range(_F)]

        @plsc.parallel_loop(0, _CSTEPS, unroll=2)
        def gather_loop(c):
            for f in range(_F):
                sl = pl.ds(f * _COLS + c * _L, _L)
                out_v[sl] = plsc.load_gather(tab_f[f], [idx_v[sl]])

        out_h = [
            pltpu.async_copy(out_v.at[pl.ds(f * _COLS, _COLS)],
                             out_hbm.at[f, pl.ds(col0, _COLS)], osem)
            for f in range(_F)
        ]
        for h in out_h:
            h.wait()

    return sc_kernel


_make_sc_kernel = functools.cache(_make_sc_kernel)


def kernel(inputs, categ_bias, moving_mean, moving_norm):
    idx_t = jnp.transpose(inputs)
    bias_flat = jnp.reshape(categ_bias, (_TAB,))
    mean_flat = jnp.reshape(moving_mean, (_F,))
    norm_flat = jnp.reshape(moving_norm, (_F,))
    out_t = _make_sc_kernel()(idx_t, bias_flat, mean_flat, norm_flat)
    return jnp.transpose(out_t)
